# baseline (speedup 1.0000x reference)
import functools

import jax
import jax.numpy as jnp
from jax import lax
from jax.experimental import pallas as pl
from jax.experimental.pallas import tpu as pltpu
from jax.experimental.pallas import tpu_sc as plsc

B = 16384
NF = 26
VOCAB = 100000
E = 16
NCONT = 74
OUT = NF * E + NCONT

NC = 2
NS = 16
NW = NC * NS
BPW = B // NW
R = 128
NCHUNK = BPW // R


@functools.partial(
    pl.kernel,
    mesh=plsc.VectorSubcoreMesh(core_axis_name="c", subcore_axis_name="s"),
    out_type=jax.ShapeDtypeStruct((B * OUT,), jnp.float32),
    compiler_params=pltpu.CompilerParams(
        use_tc_tiling_on_sc=False, needs_layout_passes=False
    ),
    scratch_types=[
        pltpu.VMEM((NF * R,), jnp.int32),
        pltpu.VMEM((NF * R, E), jnp.float32),
        pltpu.VMEM((R * NCONT,), jnp.int32),
        pltpu.VMEM((R * OUT,), jnp.float32),
        pltpu.SemaphoreType.DMA,
        pltpu.SemaphoreType.DMA,
    ],
)
def _sc_embed(xtc_hbm, xcont_hbm, w_hbm, out_hbm, idx_v, emb_v, xc_v, out_v,
              sem_in, sem_g):
    wid = lax.axis_index("s") * NC + lax.axis_index("c")
    iota = lax.iota(jnp.int32, 16)

    for c in range(NCHUNK):
        base = wid * BPW + c * R

        cps = [
            pltpu.async_copy(
                xtc_hbm.at[pl.ds(f * B + base, R)],
                idx_v.at[pl.ds(f * R, R)],
                sem_in,
            )
            for f in range(NF)
        ]
        cps.append(
            pltpu.async_copy(
                xcont_hbm.at[pl.ds(base * NCONT, R * NCONT)], xc_v, sem_in
            )
        )
        for cp in cps:
            cp.wait()

        for f in range(1, NF):
            for i in range(R // 16):
                idx_v[pl.ds(f * R + i * 16, 16)] += f * VOCAB

        gps = [
            pltpu.async_copy(
                w_hbm.at[idx_v.at[pl.ds(f * R, R)]],
                emb_v.at[pl.ds(f * R, R)],
                sem_g,
            )
            for f in range(NF)
        ]

        def cont_body(i, _):
            e = i * 16 + iota
            r = e // NCONT
            dst = (NF * E) + e + r * (OUT - NCONT)
            vals = xc_v[pl.ds(i * 16, 16)].astype(jnp.float32)
            plsc.store_scatter(out_v, [dst], vals)
            return 0

        lax.fori_loop(0, R * NCONT // 16, cont_body, 0)

        for gp in gps:
            gp.wait()

        def emb_body(i, _):
            f = i // R
            r = i - f * R
            dst = r * OUT + f * E + iota
            plsc.store_scatter(out_v, [dst], emb_v[i, :])
            return 0

        lax.fori_loop(0, NF * R, emb_body, 0)

        pltpu.sync_copy(out_v, out_hbm.at[pl.ds(base * OUT, R * OUT)])


def kernel(x, W):
    xtc = x[:, :NF].T.reshape(-1)
    xcont = x[:, NF:].reshape(-1)
    wf = W.reshape(NF * VOCAB, E)
    return _sc_embed(xtc, xcont, wf).reshape(B, OUT)
